# baseline (speedup 1.0000x reference)
import jax
import jax.numpy as jnp
from jax import lax
from jax.experimental import pallas as pl
from jax.experimental.pallas import tpu as pltpu
from jax.experimental.pallas import tpu_sc as plsc

VOCAB = 100000
EMBED = 64
BATCH = 1024
CTX = 20

NUM_CORES = 2
NUM_SUBCORES = 16
NW = NUM_CORES * NUM_SUBCORES
B_PER_W = BATCH // NW
ROWS_PER_W = B_PER_W * CTX
IDX_CHUNK = 128
N_CHUNKS = ROWS_PER_W // IDX_CHUNK

LANES = 16
VB = 1024
NV = pl.cdiv(VOCAB, VB)


def _pool_body(idx_hbm, table_hbm, out_hbm, idx_v, rows_v, pooled_v, sem):
    wid = lax.axis_index("s") * NUM_CORES + lax.axis_index("c")

    pltpu.sync_copy(idx_hbm.at[wid], idx_v)

    copies = []
    for j in range(N_CHUNKS):
        copies.append(
            pltpu.async_copy(
                table_hbm.at[idx_v.at[j]],
                rows_v.at[pl.ds(j * IDX_CHUNK, IDX_CHUNK)],
                sem,
            )
        )
    for c in copies:
        c.wait()

    inv_ctx = jnp.float32(1.0 / CTX)

    def body(b, carry):
        base = b * CTX
        for c in range(EMBED // LANES):
            sl = pl.ds(c * LANES, LANES)
            acc = rows_v[base, sl]
            for t in range(1, CTX):
                acc = acc + rows_v[base + t, sl]
            pooled_v[b, sl] = acc * inv_ctx
        return carry

    lax.fori_loop(0, B_PER_W, body, 0)

    pltpu.sync_copy(pooled_v, out_hbm.at[pl.ds(wid * B_PER_W, B_PER_W)])


def _pool(idx, table):
    mesh = plsc.VectorSubcoreMesh(
        core_axis_name="c", subcore_axis_name="s",
        num_cores=NUM_CORES, num_subcores=NUM_SUBCORES,
    )
    fn = pl.kernel(
        _pool_body,
        out_type=jax.ShapeDtypeStruct((BATCH, EMBED), jnp.float32),
        mesh=mesh,
        scratch_types=[
            pltpu.VMEM((N_CHUNKS, IDX_CHUNK), jnp.int32),
            pltpu.VMEM((ROWS_PER_W, EMBED), jnp.float32),
            pltpu.VMEM((B_PER_W, EMBED), jnp.float32),
            pltpu.SemaphoreType.DMA,
        ],
        compiler_params=pltpu.CompilerParams(use_tc_tiling_on_sc=False),
    )
    return fn(idx, table)


def _proj_body(pooled_ref, wt_ref, b_ref, out_ref):
    x = pooled_ref[...].astype(jnp.bfloat16)
    wt = wt_ref[...].astype(jnp.bfloat16)
    bias_col = jnp.transpose(b_ref[...], (1, 0))
    out_ref[...] = lax.dot_general(
        wt, x, (((0,), (1,)), ((), ())), preferred_element_type=jnp.float32
    ) + bias_col


def _proj_t(pooled, linear_w_t, linear_b):
    return pl.pallas_call(
        _proj_body,
        grid=(NV,),
        in_specs=[
            pl.BlockSpec((BATCH, EMBED), lambda j: (0, 0)),
            pl.BlockSpec((EMBED, VB), lambda j: (0, j)),
            pl.BlockSpec((1, VB), lambda j: (0, j)),
        ],
        out_specs=pl.BlockSpec((VB, BATCH), lambda j: (j, 0)),
        out_shape=jax.ShapeDtypeStruct((VOCAB, BATCH), jnp.float32),
        compiler_params=pltpu.CompilerParams(
            dimension_semantics=("parallel",),
        ),
    )(pooled, linear_w_t, linear_b.reshape(1, VOCAB))


def kernel(input_token_ids, embeddings, linear_w, linear_b):
    idx = input_token_ids.astype(jnp.int32).reshape(NW, N_CHUNKS, IDX_CHUNK)
    pooled = _pool(idx, embeddings)
    out_t = _proj_t(pooled, linear_w.T, linear_b)
    return out_t.T


# speedup vs baseline: 1.1168x; 1.1168x over previous
import jax
import jax.numpy as jnp
from jax import lax
from jax.experimental import pallas as pl
from jax.experimental.pallas import tpu as pltpu
from jax.experimental.pallas import tpu_sc as plsc

VOCAB = 100000
EMBED = 64
BATCH = 1024
CTX = 20

NUM_CORES = 2
NUM_SUBCORES = 16
NW = NUM_CORES * NUM_SUBCORES
B_PER_W = BATCH // NW
ROWS_PER_W = B_PER_W * CTX
IDX_CHUNK = 128
N_CHUNKS = ROWS_PER_W // IDX_CHUNK

LANES = 16
VB = 2048
NV = pl.cdiv(VOCAB, VB)

def _pool_body(idx_hbm, table_hbm, out_hbm, idx_v, rows_v, pooled_v, sem):
    wid = lax.axis_index("s") * NUM_CORES + lax.axis_index("c")

    pltpu.sync_copy(idx_hbm.at[wid], idx_v)

    copies = []
    for j in range(N_CHUNKS):
        copies.append(
            pltpu.async_copy(
                table_hbm.at[idx_v.at[j]],
                rows_v.at[pl.ds(j * IDX_CHUNK, IDX_CHUNK)],
                sem,
            )
        )
    for c in copies:
        c.wait()

    inv_ctx = jnp.float32(1.0 / CTX)

    def body(b, carry):
        base = b * CTX
        for c in range(EMBED // LANES):
            sl = pl.ds(c * LANES, LANES)
            acc = rows_v[base, sl]
            for t in range(1, CTX):
                acc = acc + rows_v[base + t, sl]
            pooled_v[b, sl] = acc * inv_ctx
        return carry

    lax.fori_loop(0, B_PER_W, body, 0)

    pltpu.sync_copy(pooled_v, out_hbm.at[pl.ds(wid * B_PER_W, B_PER_W)])


def _pool(idx, table):
    mesh = plsc.VectorSubcoreMesh(
        core_axis_name="c", subcore_axis_name="s",
        num_cores=NUM_CORES, num_subcores=NUM_SUBCORES,
    )
    fn = pl.kernel(
        _pool_body,
        out_type=jax.ShapeDtypeStruct((BATCH, EMBED), jnp.float32),
        mesh=mesh,
        scratch_types=[
            pltpu.VMEM((N_CHUNKS, IDX_CHUNK), jnp.int32),
            pltpu.VMEM((ROWS_PER_W, EMBED), jnp.float32),
            pltpu.VMEM((B_PER_W, EMBED), jnp.float32),
            pltpu.SemaphoreType.DMA,
        ],
        compiler_params=pltpu.CompilerParams(use_tc_tiling_on_sc=False),
    )
    return fn(idx, table)


def _proj_body(pooled_ref, wt_ref, b_ref, out_ref):
    x = pooled_ref[...].astype(jnp.bfloat16)
    wt = wt_ref[...].astype(jnp.bfloat16)
    bias_col = jnp.transpose(b_ref[...], (1, 0))
    out_ref[...] = lax.dot_general(
        wt, x, (((0,), (1,)), ((), ())), preferred_element_type=jnp.float32
    ) + bias_col


def _proj_t(pooled, linear_w_t, linear_b):
    return pl.pallas_call(
        _proj_body,
        grid=(NV,),
        in_specs=[
            pl.BlockSpec((BATCH, EMBED), lambda j: (0, 0)),
            pl.BlockSpec((EMBED, VB), lambda j: (0, j)),
            pl.BlockSpec((1, VB), lambda j: (0, j)),
        ],
        out_specs=pl.BlockSpec((VB, BATCH), lambda j: (j, 0)),
        out_shape=jax.ShapeDtypeStruct((VOCAB, BATCH), jnp.float32),
        compiler_params=pltpu.CompilerParams(
            dimension_semantics=("parallel",),
        ),
    )(pooled, linear_w_t, linear_b.reshape(1, VOCAB))


def kernel(input_token_ids, embeddings, linear_w, linear_b):
    idx = input_token_ids.astype(jnp.int32).reshape(NW, N_CHUNKS, IDX_CHUNK)
    pooled = _pool(idx, embeddings)
    out_t = _proj_t(pooled, linear_w.T, linear_b)
    return out_t.T


# speedup vs baseline: 1.1260x; 1.0082x over previous
import jax
import jax.numpy as jnp
from jax import lax
from jax.experimental import pallas as pl
from jax.experimental.pallas import tpu as pltpu
from jax.experimental.pallas import tpu_sc as plsc

VOCAB = 100000
EMBED = 64
BATCH = 1024
CTX = 20

NUM_CORES = 2
NUM_SUBCORES = 16
NW = NUM_CORES * NUM_SUBCORES
B_PER_W = BATCH // NW
ROWS_PER_W = B_PER_W * CTX
IDX_CHUNK = 128
N_CHUNKS = ROWS_PER_W // IDX_CHUNK

LANES = 16
VB = 4096
NV = pl.cdiv(VOCAB, VB)

def _pool_body(idx_hbm, table_hbm, out_hbm, idx_v, rows_v, pooled_v, sem):
    wid = lax.axis_index("s") * NUM_CORES + lax.axis_index("c")

    pltpu.sync_copy(idx_hbm.at[wid], idx_v)

    copies = []
    for j in range(N_CHUNKS):
        copies.append(
            pltpu.async_copy(
                table_hbm.at[idx_v.at[j]],
                rows_v.at[pl.ds(j * IDX_CHUNK, IDX_CHUNK)],
                sem,
            )
        )
    for c in copies:
        c.wait()

    inv_ctx = jnp.float32(1.0 / CTX)

    def body(b, carry):
        base = b * CTX
        for c in range(EMBED // LANES):
            sl = pl.ds(c * LANES, LANES)
            acc = rows_v[base, sl]
            for t in range(1, CTX):
                acc = acc + rows_v[base + t, sl]
            pooled_v[b, sl] = acc * inv_ctx
        return carry

    lax.fori_loop(0, B_PER_W, body, 0)

    pltpu.sync_copy(pooled_v, out_hbm.at[pl.ds(wid * B_PER_W, B_PER_W)])


def _pool(idx, table):
    mesh = plsc.VectorSubcoreMesh(
        core_axis_name="c", subcore_axis_name="s",
        num_cores=NUM_CORES, num_subcores=NUM_SUBCORES,
    )
    fn = pl.kernel(
        _pool_body,
        out_type=jax.ShapeDtypeStruct((BATCH, EMBED), jnp.float32),
        mesh=mesh,
        scratch_types=[
            pltpu.VMEM((N_CHUNKS, IDX_CHUNK), jnp.int32),
            pltpu.VMEM((ROWS_PER_W, EMBED), jnp.float32),
            pltpu.VMEM((B_PER_W, EMBED), jnp.float32),
            pltpu.SemaphoreType.DMA,
        ],
        compiler_params=pltpu.CompilerParams(use_tc_tiling_on_sc=False),
    )
    return fn(idx, table)


def _proj_body(pooled_ref, wt_ref, b_ref, out_ref):
    x = pooled_ref[...].astype(jnp.bfloat16)
    wt = wt_ref[...].astype(jnp.bfloat16)
    bias_col = jnp.transpose(b_ref[...], (1, 0))
    out_ref[...] = lax.dot_general(
        wt, x, (((0,), (1,)), ((), ())), preferred_element_type=jnp.float32
    ) + bias_col


def _proj_t(pooled, linear_w_t, linear_b):
    return pl.pallas_call(
        _proj_body,
        grid=(NV,),
        in_specs=[
            pl.BlockSpec((BATCH, EMBED), lambda j: (0, 0)),
            pl.BlockSpec((EMBED, VB), lambda j: (0, j)),
            pl.BlockSpec((1, VB), lambda j: (0, j)),
        ],
        out_specs=pl.BlockSpec((VB, BATCH), lambda j: (j, 0)),
        out_shape=jax.ShapeDtypeStruct((VOCAB, BATCH), jnp.float32),
        compiler_params=pltpu.CompilerParams(
            dimension_semantics=("parallel",),
        ),
    )(pooled, linear_w_t, linear_b.reshape(1, VOCAB))


def kernel(input_token_ids, embeddings, linear_w, linear_b):
    idx = input_token_ids.astype(jnp.int32).reshape(NW, N_CHUNKS, IDX_CHUNK)
    pooled = _pool(idx, embeddings)
    out_t = _proj_t(pooled, linear_w.T, linear_b)
    return out_t.T
